# baseline (speedup 1.0000x reference)
import functools

import jax
import jax.numpy as jnp
from jax import lax
from jax.experimental import pallas as pl
from jax.experimental.pallas import tpu as pltpu
from jax.experimental.pallas import tpu_sc as plsc

NC = 2
NS = 16
NW = NC * NS


def _edge_proj_body(ea_ref, we_ref, be_ref, e_ref):
    e_ref[...] = (
        jnp.dot(ea_ref[...], we_ref[...], preferred_element_type=jnp.float32)
        + be_ref[...]
    )


@functools.partial(jax.jit, static_argnames=("block",))
def _edge_proj(edge_attr, we, be, block=8192):
    E, ED = edge_attr.shape
    D = we.shape[1]
    grid = E // block
    return pl.pallas_call(
        _edge_proj_body,
        grid=(grid,),
        in_specs=[
            pl.BlockSpec((block, ED), lambda i: (i, 0)),
            pl.BlockSpec((ED, D), lambda i: (0, 0)),
            pl.BlockSpec((1, D), lambda i: (0, 0)),
        ],
        out_specs=pl.BlockSpec((block, D), lambda i: (i, 0)),
        out_shape=jax.ShapeDtypeStruct((E, D), jnp.float32),
    )(edge_attr, we, be)


def _node_body(h_ref, p_ref, w1_ref, b1_ref, w2_ref, b2_ref, g_ref, bt_ref,
               o_ref):
    N = h_ref.shape[0]
    z = h_ref[...] + p_ref[0, :N] + p_ref[1, :N]
    z = jnp.maximum(
        jnp.dot(z, w1_ref[...], preferred_element_type=jnp.float32)
        + b1_ref[...], 0.0)
    z = (jnp.dot(z, w2_ref[...], preferred_element_type=jnp.float32)
         + b2_ref[...])
    mu = jnp.mean(z, axis=0, keepdims=True)
    zc = z - mu
    var = jnp.mean(zc * zc, axis=0, keepdims=True)
    zn = zc * lax.rsqrt(var + 1e-5)
    o_ref[...] = jnp.maximum(zn * g_ref[...] + bt_ref[...], 0.0)


@jax.jit
def _node_update(h, parts, w1, b1, w2, b2, g, bt):
    N, D = h.shape
    return pl.pallas_call(
        _node_body,
        out_shape=jax.ShapeDtypeStruct((N, D), jnp.float32),
    )(h, parts, w1, b1, w2, b2, g, bt)


def _make_sc_agg(N, D, EPAD, C):
    EPW = EPAD // NW
    NCHUNK = EPW // C
    NP = 4
    assert NCHUNK * C == EPW and C % 8 == 0 and NCHUNK % 8 == 0
    assert NCHUNK >= 16
    RPT = -(-N // (NS * 8)) * 8
    NPAD = RPT * NS
    mesh = plsc.VectorSubcoreMesh(core_axis_name="c", subcore_axis_name="s",
                                  num_cores=NC)

    def body(h_hbm, src_hbm, dst_hbm, e_hbm, parts_hbm,
             agg_sh, idx_s, idx_d, hbuf, ebuf, sbuf, *sems):
        gsem = sems[0:2]
        esem = sems[2:4]
        ssem = sems[4:6]
        isem = [sems[6:8], sems[8:10], sems[10:12], sems[12:14]]
        cid = lax.axis_index("c")
        sid = lax.axis_index("s")
        wid = sid * NC + cid
        base = wid * EPW

        @pl.loop(0, C)
        def _zero_rows(r):
            for c8 in range(D // 16):
                ebuf[0, r, pl.ds(c8 * 16, 16)] = jnp.zeros((16,), jnp.float32)

        row0 = pl.multiple_of(sid * RPT, 8)
        full = RPT // C
        rem = RPT - full * C
        for k in range(full):
            pltpu.sync_copy(ebuf.at[0, pl.ds(0, C)],
                            agg_sh.at[pl.ds(row0 + k * C, C)])
        if rem:
            pltpu.sync_copy(ebuf.at[0, pl.ds(0, rem)],
                            agg_sh.at[pl.ds(row0 + full * C, rem)])
        plsc.subcore_barrier()

        def fetch_idx(j, s, q):
            pltpu.async_copy(src_hbm.at[wid, j], idx_s.at[q, s], isem[q][s])
            pltpu.async_copy(dst_hbm.at[wid, j], idx_d.at[q, s], isem[q][s])

        def wait_idx(s, q):
            pltpu.make_async_copy(src_hbm.at[wid, 0], idx_s.at[q, s],
                                  isem[q][s]).wait()
            pltpu.make_async_copy(dst_hbm.at[wid, 0], idx_d.at[q, s],
                                  isem[q][s]).wait()

        def issue_gather(s, q):
            pltpu.async_copy(h_hbm.at[idx_s.at[q, s]], hbuf.at[s], gsem[s])

        def issue_e(j, s):
            off = pl.multiple_of(base + j * C, 8)
            pltpu.async_copy(e_hbm.at[pl.ds(off, C)], ebuf.at[s], esem[s])

        def wait_in(s):
            pltpu.make_async_copy(h_hbm.at[idx_s.at[0, 0]], hbuf.at[s],
                                  gsem[s]).wait()
            pltpu.make_async_copy(e_hbm.at[pl.ds(0, C)], ebuf.at[s],
                                  esem[s]).wait()

        def compute(s):
            @pl.loop(0, C)
            def _rows(r):
                for c8 in range(D // 16):
                    sl = pl.ds(c8 * 16, 16)
                    sbuf[s, r, sl] = jnp.maximum(
                        hbuf[s, r, sl] + ebuf[s, r, sl], 0.0)

        def issue_scatter(s, q):
            pltpu.async_copy(sbuf.at[s], agg_sh.at[idx_d.at[q, s]], ssem[s],
                             add=True)

        def wait_scatter(s):
            pltpu.make_async_copy(sbuf.at[s], agg_sh.at[idx_d.at[0, 0]],
                                  ssem[s]).wait()

        def process(j, k, first=False, fetch=True, nxt=True):
            s = k % 2
            q = (k // 2) % NP
            qm1 = (k // 2 + NP - 1) % NP
            qp1 = (k // 2 + 1) % NP
            wait_in(s)
            if not first:
                wait_scatter(s)
            if fetch:
                fetch_idx(j + 6, s, qm1)
            compute(s)
            issue_scatter(s, q)
            if nxt:
                wait_idx(s, qp1)
                issue_gather(s, qp1)
                issue_e(j + 2, s)

        for k in range(6):
            fetch_idx(k, k % 2, (k // 2) % NP)
        for s in range(2):
            wait_idx(s, 0)
            issue_gather(s, 0)
            issue_e(s, s)

        for k in range(8):
            process(k, k, first=(k < 2))

        @pl.loop(1, NCHUNK // 8 - 1)
        def _oct(t):
            j0 = t * 8
            for k in range(8):
                process(j0 + k, k)

        j0 = NCHUNK - 8
        for k in range(8):
            process(j0 + k, k, fetch=(k < 2), nxt=(k < 6))
        for s in range(2):
            wait_scatter(s)

        plsc.subcore_barrier()

        pltpu.sync_copy(agg_sh.at[pl.ds(row0, RPT)],
                        parts_hbm.at[cid, pl.ds(row0, RPT)])

    return pl.kernel(
        body,
        out_type=jax.ShapeDtypeStruct((NC, NPAD, D), jnp.float32),
        mesh=mesh,
        scratch_types=[
            pltpu.VMEM_SHARED((NPAD, D), jnp.float32),
            pltpu.VMEM((NP, 2, C), jnp.int32),
            pltpu.VMEM((NP, 2, C), jnp.int32),
            pltpu.VMEM((2, C, D), jnp.float32),
            pltpu.VMEM((2, C, D), jnp.float32),
            pltpu.VMEM((2, C, D), jnp.float32),
        ] + [pltpu.SemaphoreType.DMA] * 14,
    )


def kernel(x, edge_index, edge_attr, We, be, W1, b1, W2, b2, gamma, beta):
    N, D = x.shape
    E = edge_attr.shape[0]
    L = We.shape[0]
    C = 40
    EPW = -(-E // (NW * 8 * C)) * 8 * C
    EPAD = EPW * NW
    pad = EPAD - E
    src = jnp.concatenate([edge_index[0], jnp.zeros((pad,), jnp.int32)])
    dst = jnp.concatenate([edge_index[1], jnp.full((pad,), N, jnp.int32)])
    src = src.reshape(NW, EPW // C, C)
    dst = dst.reshape(NW, EPW // C, C)
    ea_p = jnp.concatenate(
        [edge_attr, jnp.zeros((pad, edge_attr.shape[1]), jnp.float32)])
    sc_agg = _make_sc_agg(N, D, EPAD, C=C)

    h = x
    for l in range(L):
        e = _edge_proj(ea_p, We[l], be[l].reshape(1, -1))
        parts = sc_agg(h, src, dst, e)
        h = _node_update(h, parts, W1[l], b1[l].reshape(1, -1),
                         W2[l], b2[l].reshape(1, -1),
                         gamma[l].reshape(1, -1), beta[l].reshape(1, -1))
    return h
